# baseline (speedup 1.0000x reference)
import functools

import jax
import jax.numpy as jnp
from jax import lax
from jax.experimental import pallas as pl
from jax.experimental.pallas import tpu as pltpu
from jax.experimental.pallas import tpu_sc as plsc

NUM_NODES = 1000000
DIM = 32
BATCH = 16384

_INFO = plsc.get_sparse_core_info()
_NC, _NS, _L = _INFO.num_cores, _INFO.num_subcores, _INFO.num_lanes
_NW = _NC * _NS
_B_PER_SC = BATCH // _NC
_B_PER_W = BATCH // _NW


@functools.partial(
    pl.kernel,
    mesh=plsc.VectorSubcoreMesh(core_axis_name="c", subcore_axis_name="s"),
    out_type=jax.ShapeDtypeStruct((BATCH, DIM), jnp.float32),
    scratch_types=[
        pltpu.VMEM((_B_PER_W,), jnp.int32),
        pltpu.VMEM_SHARED((_B_PER_SC, DIM), jnp.float32),
        pltpu.SemaphoreType.DMA,
    ],
)
def _gather_kernel(idx_hbm, table_hbm, out_hbm, idx_v, rows_sh, sem):
    sc = lax.axis_index("c")
    tile = lax.axis_index("s")
    base = sc * _B_PER_SC + tile * _B_PER_W
    sh_base = tile * _B_PER_W
    pltpu.sync_copy(idx_hbm.at[pl.ds(base, _B_PER_W)], idx_v)

    def body(g, _):
        v = idx_v[pl.ds(g * _L, _L)]
        for k in range(_L):
            r = v[k]
            pltpu.async_copy(
                table_hbm.at[pl.ds(r, 1)],
                rows_sh.at[pl.ds(sh_base + g * _L + k, 1)],
                sem,
            )
        return ()

    lax.fori_loop(0, _B_PER_W // _L, body, ())
    pltpu.make_async_copy(
        table_hbm.at[pl.ds(0, _B_PER_W)],
        rows_sh.at[pl.ds(sh_base, _B_PER_W)],
        sem,
    ).wait()
    plsc.subcore_barrier()

    @pl.when(tile == 0)
    def _():
        pltpu.sync_copy(rows_sh, out_hbm.at[pl.ds(sc * _B_PER_SC, _B_PER_SC)])


def kernel(idx, weight):
    return _gather_kernel(idx.astype(jnp.int32), weight)


# speedup vs baseline: 1.1071x; 1.1071x over previous
import functools

import jax
import jax.numpy as jnp
from jax import lax
from jax.experimental import pallas as pl
from jax.experimental.pallas import tpu as pltpu
from jax.experimental.pallas import tpu_sc as plsc

NUM_NODES = 1000000
DIM = 32
BATCH = 16384

_INFO = plsc.get_sparse_core_info()
_NC, _NS, _L = _INFO.num_cores, _INFO.num_subcores, _INFO.num_lanes
_NW = _NC * _NS
_B_PER_W = BATCH // _NW
_NSEM = 8


@functools.partial(
    pl.kernel,
    mesh=plsc.VectorSubcoreMesh(core_axis_name="c", subcore_axis_name="s"),
    out_type=jax.ShapeDtypeStruct((BATCH, DIM), jnp.float32),
    scratch_types=[
        pltpu.VMEM((_B_PER_W,), jnp.int32),
        pltpu.VMEM((_B_PER_W, DIM), jnp.float32),
        [pltpu.SemaphoreType.DMA] * _NSEM,
    ],
)
def _gather_kernel(idx_hbm, table_hbm, out_hbm, idx_v, rows_v, sems):
    wid = lax.axis_index("s") * _NC + lax.axis_index("c")
    base = wid * _B_PER_W
    pltpu.sync_copy(idx_hbm.at[pl.ds(base, _B_PER_W)], idx_v)

    def body(g, _):
        v = idx_v[pl.ds(g * _L, _L)]
        for k in range(_L):
            r = v[k]
            pltpu.async_copy(
                table_hbm.at[pl.ds(r, 1)],
                rows_v.at[pl.ds(g * _L + k, 1)],
                sems[k % _NSEM],
            )
        return ()

    lax.fori_loop(0, _B_PER_W // _L, body, ())
    for j in range(_NSEM):
        pltpu.make_async_copy(
            table_hbm.at[pl.ds(0, _B_PER_W // _NSEM)],
            rows_v.at[pl.ds(0, _B_PER_W // _NSEM)],
            sems[j],
        ).wait()
    pltpu.sync_copy(rows_v, out_hbm.at[pl.ds(base, _B_PER_W)])


def kernel(idx, weight):
    return _gather_kernel(idx.astype(jnp.int32), weight)
